# baseline (speedup 1.0000x reference)
import functools

import jax
import jax.numpy as jnp
from jax import lax
from jax.experimental import pallas as pl
from jax.experimental.pallas import tpu as pltpu
from jax.experimental.pallas import tpu_sc as plsc

N = 1048576
D = 16
NC = 2
NS = 16
NW = NC * NS

SC_COLS = 491520
SCW = SC_COLS // NW
CH = 1024
NCH = SCW // CH

BC = 32768
TC_OFF = SC_COLS // BC


def _tc_partials(in_t, tg_t, q):
    grid = (N - SC_COLS) // BC

    def tc_body(in_ref, tg_ref, q_ref, acc_ref):
        qv = q_ref[...].astype(jnp.float32)
        m1 = qv * (2.0 - qv)
        m2 = qv * (qv - 1.0) * 0.5
        m1e = lax.broadcast_in_dim(m1, (D, BC), (1,))
        m2e = lax.broadcast_in_dim(m2, (D, BC), (1,))
        tv = tg_ref[...]
        dd = in_ref[...] - tv
        gap = (tv * m1e + m2e) * dd
        g2 = gap * gap
        g2 = g2[0:8] + g2[8:16]
        g2 = g2[0:4] + g2[4:8]
        g2 = g2[0:2] + g2[2:4]
        g2 = g2[0:1] + g2[1:2]
        h = BC
        while h > 1024:
            h //= 2
            g2 = g2[:, :h] + g2[:, h:2 * h]

        @pl.when(pl.program_id(0) == 0)
        def _():
            acc_ref[...] = jnp.zeros_like(acc_ref)

        acc_ref[...] += g2

    return pl.pallas_call(
        tc_body,
        grid=(grid,),
        in_specs=[
            pl.BlockSpec((D, BC), lambda i: (0, i + TC_OFF)),
            pl.BlockSpec((D, BC), lambda i: (0, i + TC_OFF)),
            pl.BlockSpec((BC,), lambda i: (i + TC_OFF,)),
        ],
        out_specs=pl.BlockSpec((1, 1024), lambda i: (0, 0)),
        out_shape=jax.ShapeDtypeStruct((1, 1024), jnp.float32),
    )(in_t, tg_t, q)


def _sc_partials(in_t, tg_t, q):
    mesh = plsc.VectorSubcoreMesh(core_axis_name="c", subcore_axis_name="s")

    @functools.partial(
        pl.kernel,
        out_type=jax.ShapeDtypeStruct((NW, 16), jnp.float32),
        mesh=mesh,
        scratch_types=[
            pltpu.VMEM((D, CH), jnp.float32),
            pltpu.VMEM((D, CH), jnp.float32),
            pltpu.VMEM((CH,), jnp.int32),
            pltpu.VMEM((D, CH), jnp.float32),
            pltpu.VMEM((D, CH), jnp.float32),
            pltpu.VMEM((CH,), jnp.int32),
            pltpu.VMEM((16,), jnp.float32),
            pltpu.SemaphoreType.DMA,
            pltpu.SemaphoreType.DMA,
            pltpu.SemaphoreType.DMA,
            pltpu.SemaphoreType.DMA,
            pltpu.SemaphoreType.DMA,
            pltpu.SemaphoreType.DMA,
        ],
    )
    def body(in_hbm, tg_hbm, q_hbm, out_hbm,
             in_v0, tg_v0, q_v0, in_v1, tg_v1, q_v1, acc_v,
             si0, st0, sq0, si1, st1, sq1):
        wid = lax.axis_index("s") * NC + lax.axis_index("c")
        base = wid * SCW
        bufs = ((in_v0, tg_v0, q_v0, si0, st0, sq0),
                (in_v1, tg_v1, q_v1, si1, st1, sq1))

        def descs(k, b):
            iv, tv, qv, si, st, sq = b
            c0 = base + k * CH
            return (
                pltpu.make_async_copy(in_hbm.at[:, pl.ds(c0, CH)], iv, si),
                pltpu.make_async_copy(tg_hbm.at[:, pl.ds(c0, CH)], tv, st),
                pltpu.make_async_copy(q_hbm.at[pl.ds(c0, CH)], qv, sq),
            )

        def start(k, b):
            for c in descs(k, b):
                c.start()

        def wait(k, b):
            for c in descs(k, b):
                c.wait()

        def compute(b, acc):
            iv, tv, qv = b[0], b[1], b[2]

            def col_group(g, acc):
                acc1, acc2 = acc
                c0 = g * 16
                qf = qv[pl.ds(c0, 16)].astype(jnp.float32)
                m1 = qf * (2.0 - qf)
                m2 = qf * (qf - 1.0) * 0.5
                for f in range(D):
                    ig = iv[f, pl.ds(c0, 16)]
                    tg = tv[f, pl.ds(c0, 16)]
                    dd = ig - tg
                    p = tg * dd
                    acc1 = acc1 + m1 * (p * p)
                    acc2 = acc2 + m2 * (dd * dd)
                return acc1, acc2

            return lax.fori_loop(0, CH // 16, col_group, acc)

        start(0, bufs[0])
        zero = jnp.zeros((16,), jnp.float32)

        def outer(i, acc):
            k0 = 2 * i
            wait(k0, bufs[0])
            start(k0 + 1, bufs[1])
            acc = compute(bufs[0], acc)
            wait(k0 + 1, bufs[1])

            @pl.when(k0 + 2 < NCH)
            def _():
                start(k0 + 2, bufs[0])

            return compute(bufs[1], acc)

        acc1, acc2 = lax.fori_loop(0, NCH // 2, outer, (zero, zero))
        acc_v[...] = acc1 + acc2
        pltpu.sync_copy(acc_v, out_hbm.at[wid])

    return body(in_t, tg_t, q)


def kernel(input_y, target_y, q, weights_gap, weights_l2):
    in_t = input_y.T
    tg_t = target_y.T
    sc = _sc_partials(in_t, tg_t, q)
    tc = _tc_partials(in_t, tg_t, q)
    total = jnp.sum(sc) + jnp.sum(tc)
    return total * jnp.float32(1.0 / (N * D))


# speedup vs baseline: 1.0061x; 1.0061x over previous
import functools

import jax
import jax.numpy as jnp
from jax import lax
from jax.experimental import pallas as pl
from jax.experimental.pallas import tpu as pltpu
from jax.experimental.pallas import tpu_sc as plsc

N = 1048576
D = 16
NC = 2
NS = 16
NW = NC * NS

SC_COLS = 458752
SCW = SC_COLS // NW
CH = 1024
NCH = SCW // CH

BC = 65536
TC_OFF = SC_COLS // BC


def _tc_partials(in_t, tg_t, q):
    grid = (N - SC_COLS) // BC

    def tc_body(in_ref, tg_ref, q_ref, acc_ref):
        qv = q_ref[...].astype(jnp.float32)
        m1 = qv * (2.0 - qv)
        m2 = qv * (qv - 1.0) * 0.5
        m1e = lax.broadcast_in_dim(m1, (D, BC), (1,))
        m2e = lax.broadcast_in_dim(m2, (D, BC), (1,))
        tv = tg_ref[...]
        dd = in_ref[...] - tv
        gap = (tv * m1e + m2e) * dd
        g2 = gap * gap
        g2 = g2[0:8] + g2[8:16]
        g2 = g2[0:4] + g2[4:8]
        g2 = g2[0:2] + g2[2:4]
        g2 = g2[0:1] + g2[1:2]
        h = BC
        while h > 1024:
            h //= 2
            g2 = g2[:, :h] + g2[:, h:2 * h]

        @pl.when(pl.program_id(0) == 0)
        def _():
            acc_ref[...] = jnp.zeros_like(acc_ref)

        acc_ref[...] += g2

    return pl.pallas_call(
        tc_body,
        grid=(grid,),
        in_specs=[
            pl.BlockSpec((D, BC), lambda i: (0, i + TC_OFF)),
            pl.BlockSpec((D, BC), lambda i: (0, i + TC_OFF)),
            pl.BlockSpec((BC,), lambda i: (i + TC_OFF,)),
        ],
        out_specs=pl.BlockSpec((1, 1024), lambda i: (0, 0)),
        out_shape=jax.ShapeDtypeStruct((1, 1024), jnp.float32),
    )(in_t, tg_t, q)


def _sc_partials(in_t, tg_t, q):
    mesh = plsc.VectorSubcoreMesh(core_axis_name="c", subcore_axis_name="s")

    @functools.partial(
        pl.kernel,
        out_type=jax.ShapeDtypeStruct((NW, 16), jnp.float32),
        mesh=mesh,
        scratch_types=[
            pltpu.VMEM((D, CH), jnp.float32),
            pltpu.VMEM((D, CH), jnp.float32),
            pltpu.VMEM((CH,), jnp.int32),
            pltpu.VMEM((D, CH), jnp.float32),
            pltpu.VMEM((D, CH), jnp.float32),
            pltpu.VMEM((CH,), jnp.int32),
            pltpu.VMEM((16,), jnp.float32),
            pltpu.SemaphoreType.DMA,
            pltpu.SemaphoreType.DMA,
            pltpu.SemaphoreType.DMA,
            pltpu.SemaphoreType.DMA,
            pltpu.SemaphoreType.DMA,
            pltpu.SemaphoreType.DMA,
        ],
    )
    def body(in_hbm, tg_hbm, q_hbm, out_hbm,
             in_v0, tg_v0, q_v0, in_v1, tg_v1, q_v1, acc_v,
             si0, st0, sq0, si1, st1, sq1):
        wid = lax.axis_index("s") * NC + lax.axis_index("c")
        base = wid * SCW
        bufs = ((in_v0, tg_v0, q_v0, si0, st0, sq0),
                (in_v1, tg_v1, q_v1, si1, st1, sq1))

        def descs(k, b):
            iv, tv, qv, si, st, sq = b
            c0 = base + k * CH
            return (
                pltpu.make_async_copy(in_hbm.at[:, pl.ds(c0, CH)], iv, si),
                pltpu.make_async_copy(tg_hbm.at[:, pl.ds(c0, CH)], tv, st),
                pltpu.make_async_copy(q_hbm.at[pl.ds(c0, CH)], qv, sq),
            )

        def start(k, b):
            for c in descs(k, b):
                c.start()

        def wait(k, b):
            for c in descs(k, b):
                c.wait()

        def compute(b, acc):
            iv, tv, qv = b[0], b[1], b[2]

            def col_group(g, acc):
                acc1, acc2 = acc
                c0 = g * 16
                qf = qv[pl.ds(c0, 16)].astype(jnp.float32)
                m1 = qf * (2.0 - qf)
                m2 = qf * (qf - 1.0) * 0.5
                for f in range(D):
                    ig = iv[f, pl.ds(c0, 16)]
                    tg = tv[f, pl.ds(c0, 16)]
                    dd = ig - tg
                    p = tg * dd
                    acc1 = acc1 + m1 * (p * p)
                    acc2 = acc2 + m2 * (dd * dd)
                return acc1, acc2

            return lax.fori_loop(0, CH // 16, col_group, acc)

        start(0, bufs[0])
        zero = jnp.zeros((16,), jnp.float32)

        def outer(i, acc):
            k0 = 2 * i
            wait(k0, bufs[0])
            start(k0 + 1, bufs[1])
            acc = compute(bufs[0], acc)
            wait(k0 + 1, bufs[1])

            @pl.when(k0 + 2 < NCH)
            def _():
                start(k0 + 2, bufs[0])

            return compute(bufs[1], acc)

        acc1, acc2 = lax.fori_loop(0, NCH // 2, outer, (zero, zero))
        acc_v[...] = acc1 + acc2
        pltpu.sync_copy(acc_v, out_hbm.at[wid])

    return body(in_t, tg_t, q)


def kernel(input_y, target_y, q, weights_gap, weights_l2):
    in_t = input_y.T
    tg_t = target_y.T
    sc = _sc_partials(in_t, tg_t, q)
    tc = _tc_partials(in_t, tg_t, q)
    total = jnp.sum(sc) + jnp.sum(tc)
    return total * jnp.float32(1.0 / (N * D))


# speedup vs baseline: 1.0169x; 1.0107x over previous
import functools

import jax
import jax.numpy as jnp
from jax import lax
from jax.experimental import pallas as pl
from jax.experimental.pallas import tpu as pltpu
from jax.experimental.pallas import tpu_sc as plsc

N = 1048576
D = 16
NC = 2
NS = 16
NW = NC * NS

SC_COLS = 458752
SCW = SC_COLS // NW
CH = 1024
NCH = SCW // CH

BC = 65536
TC_OFF = SC_COLS // BC


def _tc_partials(in_t, tg_t, q):
    grid = (N - SC_COLS) // BC

    def tc_body(in_ref, tg_ref, q_ref, acc_ref):
        qv = q_ref[...].astype(jnp.float32)
        m1 = qv * (2.0 - qv)
        m2 = qv * (qv - 1.0) * 0.5
        m1e = lax.broadcast_in_dim(m1, (D, BC), (1,))
        m2e = lax.broadcast_in_dim(m2, (D, BC), (1,))
        tv = tg_ref[...]
        dd = in_ref[...] - tv
        gap = (tv * m1e + m2e) * dd
        g2 = gap * gap
        g2 = g2[0:8] + g2[8:16]
        g2 = g2[0:4] + g2[4:8]
        g2 = g2[0:2] + g2[2:4]
        g2 = g2[0:1] + g2[1:2]
        h = BC
        while h > 1024:
            h //= 2
            g2 = g2[:, :h] + g2[:, h:2 * h]

        @pl.when(pl.program_id(0) == 0)
        def _():
            acc_ref[...] = jnp.zeros_like(acc_ref)

        acc_ref[...] += g2

    return pl.pallas_call(
        tc_body,
        grid=(grid,),
        in_specs=[
            pl.BlockSpec((D, BC), lambda i: (0, i + TC_OFF)),
            pl.BlockSpec((D, BC), lambda i: (0, i + TC_OFF)),
            pl.BlockSpec((BC,), lambda i: (i + TC_OFF,)),
        ],
        out_specs=pl.BlockSpec((1, 1024), lambda i: (0, 0)),
        out_shape=jax.ShapeDtypeStruct((1, 1024), jnp.float32),
    )(in_t, tg_t, q)


def _sc_partials(in_t, tg_t, q):
    mesh = plsc.VectorSubcoreMesh(core_axis_name="c", subcore_axis_name="s")

    @functools.partial(
        pl.kernel,
        out_type=jax.ShapeDtypeStruct((NW, 16), jnp.float32),
        mesh=mesh,
        scratch_types=[
            pltpu.VMEM((D, CH), jnp.float32),
            pltpu.VMEM((D, CH), jnp.float32),
            pltpu.VMEM((CH,), jnp.int32),
            pltpu.VMEM((D, CH), jnp.float32),
            pltpu.VMEM((D, CH), jnp.float32),
            pltpu.VMEM((CH,), jnp.int32),
            pltpu.VMEM((16,), jnp.float32),
            pltpu.SemaphoreType.DMA,
            pltpu.SemaphoreType.DMA,
        ],
    )
    def body(in_hbm, tg_hbm, q_hbm, out_hbm,
             in_v0, tg_v0, q_v0, in_v1, tg_v1, q_v1, acc_v,
             s0, s1):
        wid = lax.axis_index("s") * NC + lax.axis_index("c")
        base = wid * SCW
        bufs = ((in_v0, tg_v0, q_v0, s0),
                (in_v1, tg_v1, q_v1, s1))

        def descs(k, b):
            iv, tv, qv, sem = b
            c0 = base + k * CH
            return (
                pltpu.make_async_copy(in_hbm.at[:, pl.ds(c0, CH)], iv, sem),
                pltpu.make_async_copy(tg_hbm.at[:, pl.ds(c0, CH)], tv, sem),
                pltpu.make_async_copy(q_hbm.at[pl.ds(c0, CH)], qv, sem),
            )

        def start(k, b):
            for c in descs(k, b):
                c.start()

        def wait(k, b):
            for c in descs(k, b):
                c.wait()

        def compute(b, acc):
            iv, tv, qv = b[0], b[1], b[2]

            def col_group(g, acc):
                acc1, acc2 = acc
                c0 = g * 16
                qf = qv[pl.ds(c0, 16)].astype(jnp.float32)
                m1 = qf * (2.0 - qf)
                m2 = qf * (qf - 1.0) * 0.5
                for f in range(D):
                    ig = iv[f, pl.ds(c0, 16)]
                    tg = tv[f, pl.ds(c0, 16)]
                    dd = ig - tg
                    p = tg * dd
                    acc1 = acc1 + m1 * (p * p)
                    acc2 = acc2 + m2 * (dd * dd)
                return acc1, acc2

            return lax.fori_loop(0, CH // 16, col_group, acc)

        start(0, bufs[0])
        zero = jnp.zeros((16,), jnp.float32)

        def outer(i, acc):
            k0 = 2 * i
            wait(k0, bufs[0])
            start(k0 + 1, bufs[1])
            acc = compute(bufs[0], acc)
            wait(k0 + 1, bufs[1])

            @pl.when(k0 + 2 < NCH)
            def _():
                start(k0 + 2, bufs[0])

            return compute(bufs[1], acc)

        acc1, acc2 = lax.fori_loop(0, NCH // 2, outer, (zero, zero))
        acc_v[...] = acc1 + acc2
        pltpu.sync_copy(acc_v, out_hbm.at[wid])

    return body(in_t, tg_t, q)


def kernel(input_y, target_y, q, weights_gap, weights_l2):
    in_t = input_y.T
    tg_t = target_y.T
    sc = _sc_partials(in_t, tg_t, q)
    tc = _tc_partials(in_t, tg_t, q)
    total = jnp.sum(sc) + jnp.sum(tc)
    return total * jnp.float32(1.0 / (N * D))


# speedup vs baseline: 1.0188x; 1.0018x over previous
import functools

import jax
import jax.numpy as jnp
from jax import lax
from jax.experimental import pallas as pl
from jax.experimental.pallas import tpu as pltpu
from jax.experimental.pallas import tpu_sc as plsc

N = 1048576
D = 16
NC = 2
NS = 16
NW = NC * NS

SC_COLS = 458752
SCW = SC_COLS // NW
CH = 1024
NCH = SCW // CH

BC = 65536
TC_OFF = SC_COLS // BC


def _tc_partials(in_t, tg_t, q):
    grid = (N - SC_COLS) // BC

    def tc_body(in_ref, tg_ref, q_ref, acc_ref):
        qv = q_ref[...].astype(jnp.float32)
        m1 = qv * (2.0 - qv)
        m2 = qv * (qv - 1.0) * 0.5
        m1e = lax.broadcast_in_dim(m1, (D, BC), (1,))
        m2e = lax.broadcast_in_dim(m2, (D, BC), (1,))
        tv = tg_ref[...]
        dd = in_ref[...] - tv
        gap = (tv * m1e + m2e) * dd
        g2 = gap * gap
        g2 = g2[0:8] + g2[8:16]
        g2 = g2[0:4] + g2[4:8]
        g2 = g2[0:2] + g2[2:4]
        g2 = g2[0:1] + g2[1:2]
        h = BC
        while h > 1024:
            h //= 2
            g2 = g2[:, :h] + g2[:, h:2 * h]

        @pl.when(pl.program_id(0) == 0)
        def _():
            acc_ref[...] = jnp.zeros_like(acc_ref)

        acc_ref[...] += g2

    return pl.pallas_call(
        tc_body,
        grid=(grid,),
        in_specs=[
            pl.BlockSpec((D, BC), lambda i: (0, i + TC_OFF)),
            pl.BlockSpec((D, BC), lambda i: (0, i + TC_OFF)),
            pl.BlockSpec((BC,), lambda i: (i + TC_OFF,)),
        ],
        out_specs=pl.BlockSpec((1, 1024), lambda i: (0, 0)),
        out_shape=jax.ShapeDtypeStruct((1, 1024), jnp.float32),
    )(in_t, tg_t, q)


def _sc_partials(in_t, tg_t, q):
    mesh = plsc.VectorSubcoreMesh(core_axis_name="c", subcore_axis_name="s")

    @functools.partial(
        pl.kernel,
        out_type=jax.ShapeDtypeStruct((NW, 16), jnp.float32),
        mesh=mesh,
        scratch_types=[
            pltpu.VMEM((D, CH), jnp.float32),
            pltpu.VMEM((D, CH), jnp.float32),
            pltpu.VMEM((CH,), jnp.int32),
            pltpu.VMEM((D, CH), jnp.float32),
            pltpu.VMEM((D, CH), jnp.float32),
            pltpu.VMEM((CH,), jnp.int32),
            pltpu.VMEM((16,), jnp.float32),
            pltpu.SemaphoreType.DMA,
            pltpu.SemaphoreType.DMA,
        ],
    )
    def body(in_hbm, tg_hbm, q_hbm, out_hbm,
             in_v0, tg_v0, q_v0, in_v1, tg_v1, q_v1, acc_v,
             s0, s1):
        wid = lax.axis_index("s") * NC + lax.axis_index("c")
        base = wid * SCW
        bufs = ((in_v0, tg_v0, q_v0, s0),
                (in_v1, tg_v1, q_v1, s1))

        def descs(k, b):
            iv, tv, qv, sem = b
            c0 = base + k * CH
            return (
                pltpu.make_async_copy(in_hbm.at[:, pl.ds(c0, CH)], iv, sem),
                pltpu.make_async_copy(tg_hbm.at[:, pl.ds(c0, CH)], tv, sem),
                pltpu.make_async_copy(q_hbm.at[pl.ds(c0, CH)], qv, sem),
            )

        def start(k, b):
            for c in descs(k, b):
                c.start()

        def wait(k, b):
            for c in descs(k, b):
                c.wait()

        def compute(b, acc):
            iv, tv, qv = b[0], b[1], b[2]

            def col_group(g, acc):
                acc1, acc2 = acc
                c0 = g * 16
                qf = qv[pl.ds(c0, 16)].astype(jnp.float32)
                m1 = qf * (2.0 - qf)
                m2 = qf * (qf - 1.0) * 0.5
                for f in range(D):
                    ig = iv[f, pl.ds(c0, 16)]
                    tg = tv[f, pl.ds(c0, 16)]
                    dd = ig - tg
                    p = tg * dd
                    acc1 = acc1 + m1 * (p * p)
                    acc2 = acc2 + m2 * (dd * dd)
                return acc1, acc2

            return lax.fori_loop(0, CH // 16, col_group, acc)

        start(0, bufs[0])
        zero = jnp.zeros((16,), jnp.float32)

        def outer(i, acc):
            k0 = 2 * i
            wait(k0, bufs[0])
            start(k0 + 1, bufs[1])
            acc = compute(bufs[0], acc)
            wait(k0 + 1, bufs[1])

            @pl.when(k0 + 2 < NCH)
            def _():
                start(k0 + 2, bufs[0])

            return compute(bufs[1], acc)

        acc1, acc2 = lax.fori_loop(0, NCH // 2, outer, (zero, zero))
        acc_v[...] = acc1 + acc2
        pltpu.sync_copy(acc_v, out_hbm.at[wid])

    return body(in_t, tg_t, q)


def kernel(input_y, target_y, q, weights_gap, weights_l2):
    in_t = input_y.T
    tg_t = target_y.T
    tc = _tc_partials(in_t, tg_t, q)
    sc = _sc_partials(in_t, tg_t, q)
    total = jnp.sum(sc) + jnp.sum(tc)
    return total * jnp.float32(1.0 / (N * D))
